# Initial kernel scaffold; baseline (speedup 1.0000x reference)
import functools

import jax
import jax.numpy as jnp
from jax import lax
from jax.experimental import pallas as pl
from jax.experimental.pallas import tpu as pltpu
from jax.experimental.pallas import tpu_sc as plsc

N_NODES = 10000
N_EDGES = 320000
IN_FEATS = 128
HIDDEN = 64
OUT_FEATS = 32

NPAD = 10240
SINK = N_NODES
NT = 32
NSUB = 16
CHUNK = 128
K = 80
E_PAD = NT * K * CHUNK
D1 = 80
D2 = 32
BLK = 1024


def _make_sc_segsum(D):
  mesh = plsc.VectorSubcoreMesh(core_axis_name="c", subcore_axis_name="s")
  rows_per = NPAD // NSUB

  @functools.partial(
      pl.kernel,
      out_type=jax.ShapeDtypeStruct((2, NPAD, D), jnp.float32),
      mesh=mesh,
      scratch_types=[
          pltpu.VMEM((K + 1, CHUNK), jnp.int32),
          pltpu.VMEM((K, CHUNK), jnp.int32),
          pltpu.VMEM((CHUNK, D), jnp.float32),
          pltpu.VMEM((CHUNK, D), jnp.float32),
          pltpu.VMEM_SHARED((NPAD, D), jnp.float32),
          pltpu.SemaphoreType.DMA,
          pltpu.SemaphoreType.DMA,
      ],
  )
  def segsum(table_hbm, src_hbm, dst_hbm, zeros_hbm, out_hbm,
             src_v, dst_v, buf0, buf1, acc_sh, sem0, sem1):
    c = lax.axis_index("c")
    s = lax.axis_index("s")
    wid = s * 2 + c

    row0 = s * rows_per
    pltpu.sync_copy(zeros_hbm.at[pl.ds(row0, rows_per)],
                    acc_sh.at[pl.ds(row0, rows_per)])
    pltpu.sync_copy(src_hbm.at[wid], src_v)
    pltpu.sync_copy(dst_hbm.at[wid], dst_v)
    plsc.subcore_barrier()

    pltpu.async_copy(table_hbm.at[src_v.at[0]], buf0, sem0)

    def body(t, carry):
      j0 = 2 * t
      pltpu.async_copy(table_hbm.at[src_v.at[j0 + 1]], buf1, sem1)
      pltpu.make_async_copy(table_hbm.at[src_v.at[j0]], buf0, sem0).wait()
      pltpu.sync_copy(buf0, acc_sh.at[dst_v.at[j0]], add=True)
      pltpu.async_copy(table_hbm.at[src_v.at[j0 + 2]], buf0, sem0)
      pltpu.make_async_copy(table_hbm.at[src_v.at[j0 + 1]], buf1, sem1).wait()
      pltpu.sync_copy(buf1, acc_sh.at[dst_v.at[j0 + 1]], add=True)
      return carry

    lax.fori_loop(0, K // 2, body, 0)
    pltpu.make_async_copy(table_hbm.at[src_v.at[K]], buf0, sem0).wait()

    plsc.subcore_barrier()
    pltpu.sync_copy(acc_sh.at[pl.ds(row0, rows_per)],
                    out_hbm.at[c, pl.ds(row0, rows_per)])

  return segsum


_sc_segsum_d1 = _make_sc_segsum(D1)
_sc_segsum_d2 = _make_sc_segsum(D2)


def _tc1_body(x_ref, wl_ref, wr_ref, y1_ref, xr_ref):
  xb = x_ref[...]
  yl = jnp.dot(xb, wl_ref[...], preferred_element_type=jnp.float32)
  col = lax.broadcasted_iota(jnp.int32, (BLK, D1 - HIDDEN), 1)
  ones = jnp.where(col == 0, 1.0, 0.0).astype(jnp.float32)
  y1_ref[...] = jnp.concatenate([yl, ones], axis=1)
  xr_ref[...] = jnp.dot(xb, wr_ref[...], preferred_element_type=jnp.float32)


def _tc2_body(p_ref, xr_ref, b_ref, wl2_ref, wr2_ref, y2_ref, hr_ref, rc_ref):
  agg = p_ref[0] + p_ref[1]
  col = lax.broadcasted_iota(jnp.int32, (BLK, D1), 1)
  cnt = jnp.sum(jnp.where(col == HIDDEN, agg, 0.0), axis=1, keepdims=True)
  rc = 1.0 / jnp.maximum(cnt, 1.0)
  h = jnp.maximum(agg * rc + b_ref[...] + xr_ref[...], 0.0)
  y2_ref[...] = jnp.dot(h, wl2_ref[...], preferred_element_type=jnp.float32)
  hr_ref[...] = jnp.dot(h, wr2_ref[...], preferred_element_type=jnp.float32)
  rc_ref[...] = rc


def _tc3_body(p_ref, hr_ref, b_ref, rc_ref, o_ref):
  agg = p_ref[0] + p_ref[1]
  o_ref[...] = agg * rc_ref[...] + b_ref[...] + hr_ref[...]


def kernel(x, edge_index, W1l, b1, W1r, W2l, b2, W2r):
  f32 = jnp.float32
  grid = NPAD // BLK

  xp = jnp.zeros((NPAD, IN_FEATS), f32).at[:N_NODES].set(x)
  src = edge_index[0].astype(jnp.int32)
  dst = edge_index[1].astype(jnp.int32)
  pad = E_PAD - N_EDGES
  src_p = jnp.concatenate([src, jnp.zeros((pad,), jnp.int32)])
  dst_p = jnp.concatenate([dst, jnp.full((pad,), SINK, jnp.int32)])
  src3 = src_p.reshape(NT, K, CHUNK)
  src3 = jnp.concatenate([src3, jnp.zeros((NT, 1, CHUNK), jnp.int32)], axis=1)
  dst3 = dst_p.reshape(NT, K, CHUNK)
  w1l_t = W1l.T
  w1r_t = jnp.zeros((IN_FEATS, D1), f32).at[:, :HIDDEN].set(W1r.T)
  b1p = jnp.zeros((1, D1), f32).at[0, :HIDDEN].set(b1)
  w2l_t = jnp.zeros((D1, OUT_FEATS), f32).at[:HIDDEN].set(W2l.T)
  w2r_t = jnp.zeros((D1, OUT_FEATS), f32).at[:HIDDEN].set(W2r.T)
  b2p = b2.reshape(1, OUT_FEATS)
  zeros1 = jnp.zeros((NPAD, D1), f32)
  zeros2 = jnp.zeros((NPAD, D2), f32)

  y1, xr1 = pl.pallas_call(
      _tc1_body,
      grid=(grid,),
      in_specs=[
          pl.BlockSpec((BLK, IN_FEATS), lambda i: (i, 0)),
          pl.BlockSpec((IN_FEATS, HIDDEN), lambda i: (0, 0)),
          pl.BlockSpec((IN_FEATS, D1), lambda i: (0, 0)),
      ],
      out_specs=[
          pl.BlockSpec((BLK, D1), lambda i: (i, 0)),
          pl.BlockSpec((BLK, D1), lambda i: (i, 0)),
      ],
      out_shape=[
          jax.ShapeDtypeStruct((NPAD, D1), f32),
          jax.ShapeDtypeStruct((NPAD, D1), f32),
      ],
  )(xp, w1l_t, w1r_t)

  p1 = _sc_segsum_d1(y1, src3, dst3, zeros1)

  y2, hr2, rc = pl.pallas_call(
      _tc2_body,
      grid=(grid,),
      in_specs=[
          pl.BlockSpec((2, BLK, D1), lambda i: (0, i, 0)),
          pl.BlockSpec((BLK, D1), lambda i: (i, 0)),
          pl.BlockSpec((1, D1), lambda i: (0, 0)),
          pl.BlockSpec((D1, OUT_FEATS), lambda i: (0, 0)),
          pl.BlockSpec((D1, OUT_FEATS), lambda i: (0, 0)),
      ],
      out_specs=[
          pl.BlockSpec((BLK, D2), lambda i: (i, 0)),
          pl.BlockSpec((BLK, OUT_FEATS), lambda i: (i, 0)),
          pl.BlockSpec((BLK, 1), lambda i: (i, 0)),
      ],
      out_shape=[
          jax.ShapeDtypeStruct((NPAD, D2), f32),
          jax.ShapeDtypeStruct((NPAD, OUT_FEATS), f32),
          jax.ShapeDtypeStruct((NPAD, 1), f32),
      ],
  )(p1, xr1, b1p, w2l_t, w2r_t)

  p2 = _sc_segsum_d2(y2, src3, dst3, zeros2)

  out = pl.pallas_call(
      _tc3_body,
      grid=(grid,),
      in_specs=[
          pl.BlockSpec((2, BLK, D2), lambda i: (0, i, 0)),
          pl.BlockSpec((BLK, OUT_FEATS), lambda i: (i, 0)),
          pl.BlockSpec((1, OUT_FEATS), lambda i: (0, 0)),
          pl.BlockSpec((BLK, 1), lambda i: (i, 0)),
      ],
      out_specs=pl.BlockSpec((BLK, OUT_FEATS), lambda i: (i, 0)),
      out_shape=jax.ShapeDtypeStruct((NPAD, OUT_FEATS), f32),
  )(p2, hr2, b2p, rc)

  return out[:N_NODES]


# speedup vs baseline: 5.2335x; 5.2335x over previous
import functools

import jax
import jax.numpy as jnp
from jax import lax
from jax.experimental import pallas as pl
from jax.experimental.pallas import tpu as pltpu
from jax.experimental.pallas import tpu_sc as plsc

N_NODES = 10000
N_EDGES = 320000
IN_FEATS = 128
HIDDEN = 64
OUT_FEATS = 32

NPAD = 10240
SINK = N_NODES
NT = 32
NSUB = 16
CHUNK = 128
K = 80
E_PAD = NT * K * CHUNK
D1 = 80
D2 = 32
BLK = 1024


def _make_sc_segsum(D):
  mesh = plsc.VectorSubcoreMesh(core_axis_name="c", subcore_axis_name="s")
  rows_per = NPAD // NSUB

  @functools.partial(
      pl.kernel,
      out_type=jax.ShapeDtypeStruct((2, NPAD, D), jnp.float32),
      mesh=mesh,
      scratch_types=[
          pltpu.VMEM((K + 1, CHUNK), jnp.int32),
          pltpu.VMEM((K, CHUNK), jnp.int32),
          pltpu.VMEM((CHUNK, D), jnp.float32),
          pltpu.VMEM((CHUNK, D), jnp.float32),
          pltpu.VMEM_SHARED((NPAD, D), jnp.float32),
          pltpu.SemaphoreType.DMA,
          pltpu.SemaphoreType.DMA,
      ],
      compiler_params=pltpu.CompilerParams(use_tc_tiling_on_sc=False),
  )
  def segsum(table_hbm, src_hbm, dst_hbm, zeros_hbm, out_hbm,
             src_v, dst_v, buf0, buf1, acc_sh, sem0, sem1):
    c = lax.axis_index("c")
    s = lax.axis_index("s")
    wid = s * 2 + c

    row0 = s * rows_per
    pltpu.sync_copy(zeros_hbm.at[pl.ds(row0, rows_per)],
                    acc_sh.at[pl.ds(row0, rows_per)])
    pltpu.sync_copy(src_hbm.at[wid], src_v)
    pltpu.sync_copy(dst_hbm.at[wid], dst_v)
    plsc.subcore_barrier()

    pltpu.async_copy(table_hbm.at[src_v.at[0]], buf0, sem0)

    def body(t, carry):
      j0 = 2 * t
      pltpu.async_copy(table_hbm.at[src_v.at[j0 + 1]], buf1, sem1)
      pltpu.make_async_copy(table_hbm.at[src_v.at[j0]], buf0, sem0).wait()
      pltpu.sync_copy(buf0, acc_sh.at[dst_v.at[j0]], add=True)
      pltpu.async_copy(table_hbm.at[src_v.at[j0 + 2]], buf0, sem0)
      pltpu.make_async_copy(table_hbm.at[src_v.at[j0 + 1]], buf1, sem1).wait()
      pltpu.sync_copy(buf1, acc_sh.at[dst_v.at[j0 + 1]], add=True)
      return carry

    lax.fori_loop(0, K // 2, body, 0)
    pltpu.make_async_copy(table_hbm.at[src_v.at[K]], buf0, sem0).wait()

    plsc.subcore_barrier()
    pltpu.sync_copy(acc_sh.at[pl.ds(row0, rows_per)],
                    out_hbm.at[c, pl.ds(row0, rows_per)])

  return segsum


_sc_segsum_d1 = _make_sc_segsum(D1)
_sc_segsum_d2 = _make_sc_segsum(D2)


def _tc1_body(x_ref, wl_ref, wr_ref, y1_ref, xr_ref):
  xb = x_ref[...]
  yl = jnp.dot(xb, wl_ref[...], preferred_element_type=jnp.float32)
  col = lax.broadcasted_iota(jnp.int32, (BLK, D1 - HIDDEN), 1)
  ones = jnp.where(col == 0, 1.0, 0.0).astype(jnp.float32)
  y1_ref[...] = jnp.concatenate([yl, ones], axis=1)
  xr_ref[...] = jnp.dot(xb, wr_ref[...], preferred_element_type=jnp.float32)


def _tc2_body(p_ref, xr_ref, b_ref, wl2_ref, wr2_ref, y2_ref, hr_ref, rc_ref):
  agg = p_ref[0] + p_ref[1]
  col = lax.broadcasted_iota(jnp.int32, (BLK, D1), 1)
  cnt = jnp.sum(jnp.where(col == HIDDEN, agg, 0.0), axis=1, keepdims=True)
  rc = 1.0 / jnp.maximum(cnt, 1.0)
  h = jnp.maximum(agg * rc + b_ref[...] + xr_ref[...], 0.0)
  y2_ref[...] = jnp.dot(h, wl2_ref[...], preferred_element_type=jnp.float32)
  hr_ref[...] = jnp.dot(h, wr2_ref[...], preferred_element_type=jnp.float32)
  rc_ref[...] = rc


def _tc3_body(p_ref, hr_ref, b_ref, rc_ref, o_ref):
  agg = p_ref[0] + p_ref[1]
  o_ref[...] = agg * rc_ref[...] + b_ref[...] + hr_ref[...]


def kernel(x, edge_index, W1l, b1, W1r, W2l, b2, W2r):
  f32 = jnp.float32
  grid = NPAD // BLK

  xp = jnp.zeros((NPAD, IN_FEATS), f32).at[:N_NODES].set(x)
  src = edge_index[0].astype(jnp.int32)
  dst = edge_index[1].astype(jnp.int32)
  pad = E_PAD - N_EDGES
  src_p = jnp.concatenate([src, jnp.zeros((pad,), jnp.int32)])
  dst_p = jnp.concatenate([dst, jnp.full((pad,), SINK, jnp.int32)])
  src3 = src_p.reshape(NT, K, CHUNK)
  src3 = jnp.concatenate([src3, jnp.zeros((NT, 1, CHUNK), jnp.int32)], axis=1)
  dst3 = dst_p.reshape(NT, K, CHUNK)
  w1l_t = W1l.T
  w1r_t = jnp.zeros((IN_FEATS, D1), f32).at[:, :HIDDEN].set(W1r.T)
  b1p = jnp.zeros((1, D1), f32).at[0, :HIDDEN].set(b1)
  w2l_t = jnp.zeros((D1, OUT_FEATS), f32).at[:HIDDEN].set(W2l.T)
  w2r_t = jnp.zeros((D1, OUT_FEATS), f32).at[:HIDDEN].set(W2r.T)
  b2p = b2.reshape(1, OUT_FEATS)
  zeros1 = jnp.zeros((NPAD, D1), f32)
  zeros2 = jnp.zeros((NPAD, D2), f32)

  y1, xr1 = pl.pallas_call(
      _tc1_body,
      grid=(grid,),
      in_specs=[
          pl.BlockSpec((BLK, IN_FEATS), lambda i: (i, 0)),
          pl.BlockSpec((IN_FEATS, HIDDEN), lambda i: (0, 0)),
          pl.BlockSpec((IN_FEATS, D1), lambda i: (0, 0)),
      ],
      out_specs=[
          pl.BlockSpec((BLK, D1), lambda i: (i, 0)),
          pl.BlockSpec((BLK, D1), lambda i: (i, 0)),
      ],
      out_shape=[
          jax.ShapeDtypeStruct((NPAD, D1), f32),
          jax.ShapeDtypeStruct((NPAD, D1), f32),
      ],
  )(xp, w1l_t, w1r_t)

  p1 = _sc_segsum_d1(y1, src3, dst3, zeros1)

  y2, hr2, rc = pl.pallas_call(
      _tc2_body,
      grid=(grid,),
      in_specs=[
          pl.BlockSpec((2, BLK, D1), lambda i: (0, i, 0)),
          pl.BlockSpec((BLK, D1), lambda i: (i, 0)),
          pl.BlockSpec((1, D1), lambda i: (0, 0)),
          pl.BlockSpec((D1, OUT_FEATS), lambda i: (0, 0)),
          pl.BlockSpec((D1, OUT_FEATS), lambda i: (0, 0)),
      ],
      out_specs=[
          pl.BlockSpec((BLK, D2), lambda i: (i, 0)),
          pl.BlockSpec((BLK, OUT_FEATS), lambda i: (i, 0)),
          pl.BlockSpec((BLK, 1), lambda i: (i, 0)),
      ],
      out_shape=[
          jax.ShapeDtypeStruct((NPAD, D2), f32),
          jax.ShapeDtypeStruct((NPAD, OUT_FEATS), f32),
          jax.ShapeDtypeStruct((NPAD, 1), f32),
      ],
  )(p1, xr1, b1p, w2l_t, w2r_t)

  p2 = _sc_segsum_d2(y2, src3, dst3, zeros2)

  out = pl.pallas_call(
      _tc3_body,
      grid=(grid,),
      in_specs=[
          pl.BlockSpec((2, BLK, D2), lambda i: (0, i, 0)),
          pl.BlockSpec((BLK, OUT_FEATS), lambda i: (i, 0)),
          pl.BlockSpec((1, OUT_FEATS), lambda i: (0, 0)),
          pl.BlockSpec((BLK, 1), lambda i: (i, 0)),
      ],
      out_specs=pl.BlockSpec((BLK, OUT_FEATS), lambda i: (i, 0)),
      out_shape=jax.ShapeDtypeStruct((NPAD, OUT_FEATS), f32),
  )(p2, hr2, b2p, rc)

  return out[:N_NODES]


# speedup vs baseline: 9.2300x; 1.7636x over previous
import functools

import jax
import jax.numpy as jnp
from jax import lax
from jax.experimental import pallas as pl
from jax.experimental.pallas import tpu as pltpu
from jax.experimental.pallas import tpu_sc as plsc

N_NODES = 10000
N_EDGES = 320000
IN_FEATS = 128
HIDDEN = 64
OUT_FEATS = 32

NPAD = 10240
SINK = N_NODES
NSUB = 16
CHUNK = 128
EPT = 20480
E_PAD = NSUB * EPT
DH1 = 40
DH2 = 16
BLK = 1024


def _make_sc_segsum(D, kk, spmem_table=True):
  mesh = plsc.VectorSubcoreMesh(core_axis_name="c", subcore_axis_name="s")
  rows_per = NPAD // NSUB
  SLAB = kk * CHUNK
  G = EPT // SLAB

  @functools.partial(
      pl.kernel,
      out_type=jax.ShapeDtypeStruct((2, NPAD, D), jnp.float32),
      mesh=mesh,
      scratch_types=[
          pltpu.VMEM((G + 1, SLAB), jnp.int32),
          pltpu.VMEM((G, SLAB), jnp.int32),
          pltpu.VMEM((SLAB, D), jnp.float32),
          pltpu.VMEM((SLAB, D), jnp.float32),
          (pltpu.VMEM_SHARED((NPAD, D), jnp.float32) if spmem_table
           else pltpu.VMEM((8, CHUNK), jnp.int32)),
          pltpu.VMEM_SHARED((NPAD, D), jnp.float32),
          pltpu.SemaphoreType.DMA,
          pltpu.SemaphoreType.DMA,
      ],
      compiler_params=pltpu.CompilerParams(use_tc_tiling_on_sc=False),
  )
  def segsum(table_hbm, src_hbm, dst_hbm, zeros_hbm, out_hbm,
             src_v, dst_v, buf0, buf1, tbl_sh, acc_sh, sem0, sem1):
    c = lax.axis_index("c")
    s = lax.axis_index("s")

    row0 = s * rows_per
    if spmem_table:
      pltpu.sync_copy(table_hbm.at[c, pl.ds(row0, rows_per)],
                      tbl_sh.at[pl.ds(row0, rows_per)])
    pltpu.sync_copy(zeros_hbm.at[pl.ds(row0, rows_per)],
                    acc_sh.at[pl.ds(row0, rows_per)])
    if spmem_table:
      pltpu.sync_copy(src_hbm.at[s], src_v)
    else:
      pltpu.sync_copy(src_hbm.at[c, s], src_v)
    pltpu.sync_copy(dst_hbm.at[s], dst_v)
    plsc.subcore_barrier()

    def tbl_at(idx):
      if spmem_table:
        return tbl_sh.at[idx]
      return table_hbm.at[idx]

    pltpu.async_copy(tbl_at(src_v.at[0]), buf0, sem0)

    def body(t, carry):
      g0 = 2 * t
      pltpu.async_copy(tbl_at(src_v.at[g0 + 1]), buf1, sem1)
      pltpu.make_async_copy(tbl_at(src_v.at[g0]), buf0, sem0).wait()
      pltpu.sync_copy(buf0, acc_sh.at[dst_v.at[g0]], add=True)
      pltpu.async_copy(tbl_at(src_v.at[g0 + 2]), buf0, sem0)
      pltpu.make_async_copy(tbl_at(src_v.at[g0 + 1]), buf1, sem1).wait()
      pltpu.sync_copy(buf1, acc_sh.at[dst_v.at[g0 + 1]], add=True)
      return carry

    lax.fori_loop(0, G // 2, body, 0)
    pltpu.make_async_copy(tbl_at(src_v.at[G]), buf0, sem0).wait()

    plsc.subcore_barrier()
    pltpu.sync_copy(acc_sh.at[pl.ds(row0, rows_per)],
                    out_hbm.at[c, pl.ds(row0, rows_per)])

  return segsum


_sc_segsum_l1 = _make_sc_segsum(DH1, 1)
_sc_segsum_l2 = _make_sc_segsum(DH2, 10, spmem_table=False)


def _tc1_body(x_ref, wl_ref, wr_ref, y1_ref, xr_ref):
  xb = x_ref[...]
  yl = jnp.dot(xb, wl_ref[...], preferred_element_type=jnp.float32)
  col = lax.broadcasted_iota(jnp.int32, (BLK, DH1 - (HIDDEN - DH1)), 1)
  cnt_col = jnp.where(col == 0, 1.0, 0.0).astype(jnp.float32)
  y1_ref[0] = yl[:, :DH1]
  y1_ref[1] = jnp.concatenate([yl[:, DH1:], cnt_col], axis=1)
  xr_ref[...] = jnp.dot(xb, wr_ref[...], preferred_element_type=jnp.float32)


def _tc2_body(p_ref, xr_ref, b_ref, wl2_ref, wr2_ref, y2_ref, hr_ref, rc_ref):
  agg = jnp.concatenate([p_ref[0], p_ref[1]], axis=1)
  col = lax.broadcasted_iota(jnp.int32, (BLK, 80), 1)
  cnt = jnp.sum(jnp.where(col == HIDDEN, agg, 0.0), axis=1, keepdims=True)
  rc = 1.0 / jnp.maximum(cnt, 1.0)
  h = jnp.maximum(agg * rc + b_ref[...] + xr_ref[...], 0.0)
  y2 = jnp.dot(h, wl2_ref[...], preferred_element_type=jnp.float32)
  y2_ref[0] = y2[:, :DH2]
  y2_ref[1] = y2[:, DH2:]
  hr_ref[...] = jnp.dot(h, wr2_ref[...], preferred_element_type=jnp.float32)
  rc_ref[...] = rc


def _tc3_body(p_ref, hr_ref, b_ref, rc_ref, o_ref):
  agg = jnp.concatenate([p_ref[0], p_ref[1]], axis=1)
  o_ref[...] = agg * rc_ref[...] + b_ref[...] + hr_ref[...]


def kernel(x, edge_index, W1l, b1, W1r, W2l, b2, W2r):
  f32 = jnp.float32
  grid = NPAD // BLK

  xp = jnp.zeros((NPAD, IN_FEATS), f32).at[:N_NODES].set(x)
  src = edge_index[0].astype(jnp.int32)
  dst = edge_index[1].astype(jnp.int32)
  pad = E_PAD - N_EDGES
  src_p = jnp.concatenate([src, jnp.zeros((pad,), jnp.int32)])
  sink = SINK + jnp.arange(pad, dtype=jnp.int32) % (NPAD - SINK)
  dst_p = jnp.concatenate([dst, sink])
  src2 = src_p.reshape(NSUB, EPT)
  dst2 = dst_p.reshape(NSUB, EPT)

  def slabify(kk):
    slab = kk * CHUNK
    g = EPT // slab
    srcx = jnp.concatenate([src2, jnp.zeros((NSUB, slab), jnp.int32)], axis=1)
    return (srcx.reshape(NSUB, g + 1, slab), dst2.reshape(NSUB, g, slab))

  src3a, dst3a = slabify(1)
  src3b0, dst3b = slabify(10)
  src3b = jnp.stack([src3b0, src3b0 + NPAD])
  w1l_t = W1l.T
  w1r_t = jnp.zeros((IN_FEATS, 80), f32).at[:, :HIDDEN].set(W1r.T)
  b1p = jnp.zeros((1, 80), f32).at[0, :HIDDEN].set(b1)
  w2l_t = jnp.zeros((80, OUT_FEATS), f32).at[:HIDDEN].set(W2l.T)
  w2r_t = jnp.zeros((80, OUT_FEATS), f32).at[:HIDDEN].set(W2r.T)
  b2p = b2.reshape(1, OUT_FEATS)
  zeros1 = jnp.zeros((NPAD, DH1), f32)
  zeros2 = jnp.zeros((NPAD, DH2), f32)

  y1, xr1 = pl.pallas_call(
      _tc1_body,
      grid=(grid,),
      in_specs=[
          pl.BlockSpec((BLK, IN_FEATS), lambda i: (i, 0)),
          pl.BlockSpec((IN_FEATS, HIDDEN), lambda i: (0, 0)),
          pl.BlockSpec((IN_FEATS, 80), lambda i: (0, 0)),
      ],
      out_specs=[
          pl.BlockSpec((2, BLK, DH1), lambda i: (0, i, 0)),
          pl.BlockSpec((BLK, 80), lambda i: (i, 0)),
      ],
      out_shape=[
          jax.ShapeDtypeStruct((2, NPAD, DH1), f32),
          jax.ShapeDtypeStruct((NPAD, 80), f32),
      ],
  )(xp, w1l_t, w1r_t)

  p1 = _sc_segsum_l1(y1, src3a, dst3a, zeros1)

  y2, hr2, rc = pl.pallas_call(
      _tc2_body,
      grid=(grid,),
      in_specs=[
          pl.BlockSpec((2, BLK, DH1), lambda i: (0, i, 0)),
          pl.BlockSpec((BLK, 80), lambda i: (i, 0)),
          pl.BlockSpec((1, 80), lambda i: (0, 0)),
          pl.BlockSpec((80, OUT_FEATS), lambda i: (0, 0)),
          pl.BlockSpec((80, OUT_FEATS), lambda i: (0, 0)),
      ],
      out_specs=[
          pl.BlockSpec((2, BLK, DH2), lambda i: (0, i, 0)),
          pl.BlockSpec((BLK, OUT_FEATS), lambda i: (i, 0)),
          pl.BlockSpec((BLK, 1), lambda i: (i, 0)),
      ],
      out_shape=[
          jax.ShapeDtypeStruct((2, NPAD, DH2), f32),
          jax.ShapeDtypeStruct((NPAD, OUT_FEATS), f32),
          jax.ShapeDtypeStruct((NPAD, 1), f32),
      ],
  )(p1, xr1, b1p, w2l_t, w2r_t)

  p2 = _sc_segsum_l2(y2.reshape(2 * NPAD, DH2), src3b, dst3b, zeros2)

  out = pl.pallas_call(
      _tc3_body,
      grid=(grid,),
      in_specs=[
          pl.BlockSpec((2, BLK, DH2), lambda i: (0, i, 0)),
          pl.BlockSpec((BLK, OUT_FEATS), lambda i: (i, 0)),
          pl.BlockSpec((1, OUT_FEATS), lambda i: (0, 0)),
          pl.BlockSpec((BLK, 1), lambda i: (i, 0)),
      ],
      out_specs=pl.BlockSpec((BLK, OUT_FEATS), lambda i: (i, 0)),
      out_shape=jax.ShapeDtypeStruct((NPAD, OUT_FEATS), f32),
  )(p2, hr2, b2p, rc)

  return out[:N_NODES]


# speedup vs baseline: 12.2927x; 1.3318x over previous
import functools

import jax
import jax.numpy as jnp
from jax import lax
from jax.experimental import pallas as pl
from jax.experimental.pallas import tpu as pltpu
from jax.experimental.pallas import tpu_sc as plsc

N_NODES = 10000
N_EDGES = 320000
IN_FEATS = 128
HIDDEN = 64
OUT_FEATS = 32

NPAD = 10240
SINK = N_NODES
NSUB = 16
CHUNK = 128
EPT = 20480
E_PAD = NSUB * EPT
DH1 = 40
DH2 = 16
BLK = 1024
ROWS_PER = NPAD // NSUB
MESH = dict(core_axis_name="c", subcore_axis_name="s")


G1 = EPT // CHUNK
NB = 4


@functools.partial(
    pl.kernel,
    out_type=jax.ShapeDtypeStruct((2, NPAD, DH1), jnp.float32),
    mesh=plsc.VectorSubcoreMesh(**MESH),
    scratch_types=[
        pltpu.VMEM((G1 + NB, CHUNK), jnp.int32),
        pltpu.VMEM((G1, CHUNK), jnp.int32),
        [pltpu.VMEM((CHUNK, DH1), jnp.float32)] * NB,
        pltpu.VMEM_SHARED((NPAD, DH1), jnp.float32),
        pltpu.VMEM_SHARED((NPAD, DH1), jnp.float32),
        [pltpu.SemaphoreType.DMA] * NB,
        [pltpu.SemaphoreType.DMA] * NB,
    ],
    compiler_params=pltpu.CompilerParams(use_tc_tiling_on_sc=False),
)
def _sc_segsum_l1(table_hbm, src_hbm, dst_hbm, zeros_hbm, out_hbm,
                  src_v, dst_v, bufs, tbl_sh, acc_sh, gsem, ssem):
  c = lax.axis_index("c")
  s = lax.axis_index("s")

  row0 = s * ROWS_PER
  pltpu.sync_copy(table_hbm.at[c, pl.ds(row0, ROWS_PER)],
                  tbl_sh.at[pl.ds(row0, ROWS_PER)])
  pltpu.sync_copy(zeros_hbm.at[pl.ds(row0, ROWS_PER)],
                  acc_sh.at[pl.ds(row0, ROWS_PER)])
  pltpu.sync_copy(src_hbm.at[s], src_v)
  pltpu.sync_copy(dst_hbm.at[s], dst_v)
  plsc.subcore_barrier()

  for i in range(NB):
    pltpu.async_copy(tbl_sh.at[src_v.at[i]], bufs[i], gsem[i])

  def body(t, carry):
    j0 = NB * t
    scs = []
    for i in range(NB):
      pltpu.make_async_copy(tbl_sh.at[src_v.at[j0 + i]],
                            bufs[i], gsem[i]).wait()
      scs.append(pltpu.async_copy(bufs[i], acc_sh.at[dst_v.at[j0 + i]],
                                  ssem[i], add=True))
    for i in range(NB):
      scs[i].wait()
      pltpu.async_copy(tbl_sh.at[src_v.at[j0 + NB + i]], bufs[i], gsem[i])
    return carry

  lax.fori_loop(0, G1 // NB, body, 0)
  for i in range(NB):
    pltpu.make_async_copy(tbl_sh.at[src_v.at[G1 + i]], bufs[i],
                          gsem[i]).wait()

  plsc.subcore_barrier()
  pltpu.sync_copy(acc_sh.at[pl.ds(row0, ROWS_PER)],
                  out_hbm.at[c, pl.ds(row0, ROWS_PER)])


KK2 = 10
SLAB2 = KK2 * CHUNK
G2 = EPT // SLAB2


@functools.partial(
    pl.kernel,
    out_type=jax.ShapeDtypeStruct((2, NPAD, DH2), jnp.float32),
    mesh=plsc.VectorSubcoreMesh(**MESH),
    scratch_types=[
        pltpu.VMEM((G2 + 1, SLAB2), jnp.int32),
        pltpu.VMEM((G2, SLAB2), jnp.int32),
        pltpu.VMEM((SLAB2, DH2), jnp.float32),
        pltpu.VMEM((SLAB2, DH2), jnp.float32),
        pltpu.VMEM_SHARED((NPAD, DH2), jnp.float32),
        pltpu.VMEM_SHARED((NPAD, DH2), jnp.float32),
        pltpu.VMEM((ROWS_PER,), jnp.float32),
        pltpu.VMEM((ROWS_PER, DH2), jnp.float32),
        pltpu.VMEM((DH2,), jnp.float32),
        pltpu.VMEM((ROWS_PER, DH2), jnp.float32),
        pltpu.SemaphoreType.DMA,
        pltpu.SemaphoreType.DMA,
    ],
    compiler_params=pltpu.CompilerParams(use_tc_tiling_on_sc=False),
)
def _sc_segsum_l2(table_hbm, src_hbm, dst_hbm, zeros_hbm, rc_hbm, hr_hbm,
                  b2_hbm, out_hbm,
                  src_v, dst_v, buf0, buf1, tbl_sh, acc_sh,
                  rc_v, hr_v, b2_v, out_v, sem0, sem1):
  c = lax.axis_index("c")
  s = lax.axis_index("s")

  row0 = s * ROWS_PER
  pltpu.sync_copy(table_hbm.at[c, pl.ds(row0, ROWS_PER)],
                  tbl_sh.at[pl.ds(row0, ROWS_PER)])
  pltpu.sync_copy(zeros_hbm.at[pl.ds(row0, ROWS_PER)],
                  acc_sh.at[pl.ds(row0, ROWS_PER)])
  pltpu.sync_copy(src_hbm.at[s], src_v)
  pltpu.sync_copy(dst_hbm.at[s], dst_v)
  plsc.subcore_barrier()

  pltpu.async_copy(tbl_sh.at[src_v.at[0]], buf0, sem0)

  def body(t, carry):
    g0 = 2 * t
    pltpu.async_copy(tbl_sh.at[src_v.at[g0 + 1]], buf1, sem1)
    pltpu.make_async_copy(tbl_sh.at[src_v.at[g0]], buf0, sem0).wait()
    pltpu.sync_copy(buf0, acc_sh.at[dst_v.at[g0]], add=True)
    pltpu.async_copy(tbl_sh.at[src_v.at[g0 + 2]], buf0, sem0)
    pltpu.make_async_copy(tbl_sh.at[src_v.at[g0 + 1]], buf1, sem1).wait()
    pltpu.sync_copy(buf1, acc_sh.at[dst_v.at[g0 + 1]], add=True)
    return carry

  lax.fori_loop(0, G2 // 2, body, 0)
  pltpu.make_async_copy(tbl_sh.at[src_v.at[G2]], buf0, sem0).wait()
  plsc.subcore_barrier()

  pltpu.sync_copy(rc_hbm.at[pl.ds(row0, ROWS_PER)], rc_v)
  pltpu.sync_copy(hr_hbm.at[c, pl.ds(row0, ROWS_PER)], hr_v)
  pltpu.sync_copy(b2_hbm.at[c], b2_v)
  pltpu.sync_copy(acc_sh.at[pl.ds(row0, ROWS_PER)], out_v)
  b2vec = b2_v[...]

  def eblk(q, carry):
    rc16 = rc_v[pl.ds(q * 16, 16)]
    for i in range(16):
      r = q * 16 + i
      out_v[r, :] = out_v[r, :] * rc16[i] + b2vec + hr_v[r, :]
    return carry

  lax.fori_loop(0, ROWS_PER // 16, eblk, 0)
  pltpu.sync_copy(out_v, out_hbm.at[c, pl.ds(row0, ROWS_PER)])


def _tc1_body(x_ref, wl_ref, wr_ref, y1_ref, xr_ref):
  xb = x_ref[...]
  yl = jnp.dot(xb, wl_ref[...], preferred_element_type=jnp.float32)
  col = lax.broadcasted_iota(jnp.int32, (BLK, 2 * DH1 - HIDDEN), 1)
  cnt_col = jnp.where(col == 0, 1.0, 0.0).astype(jnp.float32)
  y1_ref[0] = yl[:, :DH1]
  y1_ref[1] = jnp.concatenate([yl[:, DH1:], cnt_col], axis=1)
  xr_ref[...] = jnp.dot(xb, wr_ref[...], preferred_element_type=jnp.float32)


def _tc2_body(p_ref, xr_ref, b_ref, wl2_ref, wr2_ref, y2_ref, hr_ref, rc_ref):
  agg = jnp.concatenate([p_ref[0], p_ref[1]], axis=1)
  col = lax.broadcasted_iota(jnp.int32, (BLK, 2 * DH1), 1)
  cnt = jnp.sum(jnp.where(col == HIDDEN, agg, 0.0), axis=1, keepdims=True)
  rc = 1.0 / jnp.maximum(cnt, 1.0)
  h = jnp.maximum(agg * rc + b_ref[...] + xr_ref[...], 0.0)
  y2 = jnp.dot(h, wl2_ref[...], preferred_element_type=jnp.float32)
  y2_ref[0] = y2[:, :DH2]
  y2_ref[1] = y2[:, DH2:]
  hr = jnp.dot(h, wr2_ref[...], preferred_element_type=jnp.float32)
  hr_ref[0] = hr[:, :DH2]
  hr_ref[1] = hr[:, DH2:]
  rc_ref[...] = rc


def kernel(x, edge_index, W1l, b1, W1r, W2l, b2, W2r):
  f32 = jnp.float32
  grid = NPAD // BLK

  xp = jnp.zeros((NPAD, IN_FEATS), f32).at[:N_NODES].set(x)
  src = edge_index[0].astype(jnp.int32)
  dst = edge_index[1].astype(jnp.int32)
  pad = E_PAD - N_EDGES
  src_p = jnp.concatenate([src, jnp.zeros((pad,), jnp.int32)])
  sink = SINK + jnp.arange(pad, dtype=jnp.int32) % (NPAD - SINK)
  dst_p = jnp.concatenate([dst, sink])
  src2 = src_p.reshape(NSUB, EPT)
  dst2 = dst_p.reshape(NSUB, EPT)

  def slabify(slab, extra):
    g = EPT // slab
    srcx = jnp.concatenate(
        [src2, jnp.zeros((NSUB, extra * slab), jnp.int32)], axis=1)
    return (srcx.reshape(NSUB, g + extra, slab), dst2.reshape(NSUB, g, slab))

  src3a, dst3a = slabify(CHUNK, NB)
  src3b, dst3b = slabify(SLAB2, 1)
  w1l_t = W1l.T
  w1r_t = jnp.zeros((IN_FEATS, 80), f32).at[:, :HIDDEN].set(W1r.T)
  b1p = jnp.zeros((1, 80), f32).at[0, :HIDDEN].set(b1)
  w2l_t = jnp.zeros((80, OUT_FEATS), f32).at[:HIDDEN].set(W2l.T)
  w2r_t = jnp.zeros((80, OUT_FEATS), f32).at[:HIDDEN].set(W2r.T)
  b2p = b2.reshape(2, DH2)
  zeros1 = jnp.zeros((NPAD, DH1), f32)
  zeros2 = jnp.zeros((NPAD, DH2), f32)

  y1, xr1 = pl.pallas_call(
      _tc1_body,
      grid=(grid,),
      in_specs=[
          pl.BlockSpec((BLK, IN_FEATS), lambda i: (i, 0)),
          pl.BlockSpec((IN_FEATS, HIDDEN), lambda i: (0, 0)),
          pl.BlockSpec((IN_FEATS, 80), lambda i: (0, 0)),
      ],
      out_specs=[
          pl.BlockSpec((2, BLK, DH1), lambda i: (0, i, 0)),
          pl.BlockSpec((BLK, 80), lambda i: (i, 0)),
      ],
      out_shape=[
          jax.ShapeDtypeStruct((2, NPAD, DH1), f32),
          jax.ShapeDtypeStruct((NPAD, 80), f32),
      ],
  )(xp, w1l_t, w1r_t)

  p1 = _sc_segsum_l1(y1, src3a, dst3a, zeros1)

  y2, hr2, rc = pl.pallas_call(
      _tc2_body,
      grid=(grid,),
      in_specs=[
          pl.BlockSpec((2, BLK, DH1), lambda i: (0, i, 0)),
          pl.BlockSpec((BLK, 80), lambda i: (i, 0)),
          pl.BlockSpec((1, 80), lambda i: (0, 0)),
          pl.BlockSpec((80, OUT_FEATS), lambda i: (0, 0)),
          pl.BlockSpec((80, OUT_FEATS), lambda i: (0, 0)),
      ],
      out_specs=[
          pl.BlockSpec((2, BLK, DH2), lambda i: (0, i, 0)),
          pl.BlockSpec((2, BLK, DH2), lambda i: (0, i, 0)),
          pl.BlockSpec((BLK, 1), lambda i: (i, 0)),
      ],
      out_shape=[
          jax.ShapeDtypeStruct((2, NPAD, DH2), f32),
          jax.ShapeDtypeStruct((2, NPAD, DH2), f32),
          jax.ShapeDtypeStruct((NPAD, 1), f32),
      ],
  )(p1, xr1, b1p, w2l_t, w2r_t)

  p2 = _sc_segsum_l2(y2, src3b, dst3b, zeros2, rc.reshape(NPAD), hr2, b2p)

  return jnp.concatenate([p2[0], p2[1]], axis=1)[:N_NODES]


# speedup vs baseline: 13.4317x; 1.0927x over previous
import functools

import jax
import jax.numpy as jnp
from jax import lax
from jax.experimental import pallas as pl
from jax.experimental.pallas import tpu as pltpu
from jax.experimental.pallas import tpu_sc as plsc

N_NODES = 10000
N_EDGES = 320000
IN_FEATS = 128
HIDDEN = 64
OUT_FEATS = 32

NPAD = 10240
SINK = N_NODES
NSUB = 16
CHUNK = 128
EPT = 20480
E_PAD = NSUB * EPT
D1 = 80
DH1 = 40
D2 = 32
DH2 = 16
BLK = 1024
ROWS_PER = NPAD // NSUB
MESH = dict(core_axis_name="c", subcore_axis_name="s")

G1 = EPT // CHUNK
NB = 4
KK2 = 10
SLAB2 = KK2 * CHUNK
G2 = EPT // SLAB2


@functools.partial(
    pl.kernel,
    out_type=jax.ShapeDtypeStruct((NPAD, D1), jnp.float32),
    mesh=plsc.VectorSubcoreMesh(**MESH),
    scratch_types=[
        pltpu.VMEM((G1 + NB, CHUNK), jnp.int32),
        pltpu.VMEM((G1, CHUNK), jnp.int32),
        [pltpu.VMEM((CHUNK, DH1), jnp.float32)] * NB,
        pltpu.VMEM_SHARED((NPAD, DH1), jnp.float32),
        pltpu.VMEM_SHARED((NPAD, DH1), jnp.float32),
        [pltpu.SemaphoreType.DMA] * NB,
        [pltpu.SemaphoreType.DMA] * NB,
    ],
    compiler_params=pltpu.CompilerParams(use_tc_tiling_on_sc=False),
)
def _sc_segsum_l1(table_hbm, src_hbm, dst_hbm, zeros_hbm, out_hbm,
                  src_v, dst_v, bufs, tbl_sh, acc_sh, gsem, ssem):
  c = lax.axis_index("c")
  s = lax.axis_index("s")

  row0 = s * ROWS_PER
  col0 = c * DH1
  pltpu.sync_copy(table_hbm.at[pl.ds(row0, ROWS_PER), pl.ds(col0, DH1)],
                  tbl_sh.at[pl.ds(row0, ROWS_PER)])
  pltpu.sync_copy(zeros_hbm.at[pl.ds(row0, ROWS_PER)],
                  acc_sh.at[pl.ds(row0, ROWS_PER)])
  pltpu.sync_copy(src_hbm.at[s, pl.ds(0, G1 + NB)], src_v)
  pltpu.sync_copy(dst_hbm.at[s], dst_v)
  plsc.subcore_barrier()

  for i in range(NB):
    pltpu.async_copy(tbl_sh.at[src_v.at[i]], bufs[i], gsem[i])

  def body(t, carry):
    j0 = NB * t
    scs = []
    for i in range(NB):
      pltpu.make_async_copy(tbl_sh.at[src_v.at[j0 + i]],
                            bufs[i], gsem[i]).wait()
      scs.append(pltpu.async_copy(bufs[i], acc_sh.at[dst_v.at[j0 + i]],
                                  ssem[i], add=True))
    for i in range(NB):
      scs[i].wait()
      pltpu.async_copy(tbl_sh.at[src_v.at[j0 + NB + i]], bufs[i], gsem[i])
    return carry

  lax.fori_loop(0, G1 // NB, body, 0)
  for i in range(NB):
    pltpu.make_async_copy(tbl_sh.at[src_v.at[G1 + i]], bufs[i],
                          gsem[i]).wait()

  plsc.subcore_barrier()
  pltpu.sync_copy(acc_sh.at[pl.ds(row0, ROWS_PER)],
                  out_hbm.at[pl.ds(row0, ROWS_PER), pl.ds(col0, DH1)])


@functools.partial(
    pl.kernel,
    out_type=jax.ShapeDtypeStruct((NPAD, D2), jnp.float32),
    mesh=plsc.VectorSubcoreMesh(**MESH),
    scratch_types=[
        pltpu.VMEM((G2 + 1, SLAB2), jnp.int32),
        pltpu.VMEM((G2, SLAB2), jnp.int32),
        pltpu.VMEM((SLAB2, DH2), jnp.float32),
        pltpu.VMEM((SLAB2, DH2), jnp.float32),
        pltpu.VMEM_SHARED((NPAD, DH2), jnp.float32),
        pltpu.VMEM_SHARED((NPAD, DH2), jnp.float32),
        pltpu.VMEM((ROWS_PER,), jnp.float32),
        pltpu.VMEM((ROWS_PER, DH2), jnp.float32),
        pltpu.VMEM((DH2,), jnp.float32),
        pltpu.VMEM((ROWS_PER, DH2), jnp.float32),
        pltpu.SemaphoreType.DMA,
        pltpu.SemaphoreType.DMA,
    ],
    compiler_params=pltpu.CompilerParams(use_tc_tiling_on_sc=False),
)
def _sc_segsum_l2(table_hbm, src_hbm, dst_hbm, zeros_hbm, rc_hbm, hr_hbm,
                  b2_hbm, out_hbm,
                  src_v, dst_v, buf0, buf1, tbl_sh, acc_sh,
                  rc_v, hr_v, b2_v, out_v, sem0, sem1):
  c = lax.axis_index("c")
  s = lax.axis_index("s")

  row0 = s * ROWS_PER
  col0 = c * DH2
  pltpu.sync_copy(table_hbm.at[pl.ds(row0, ROWS_PER), pl.ds(col0, DH2)],
                  tbl_sh.at[pl.ds(row0, ROWS_PER)])
  pltpu.sync_copy(zeros_hbm.at[pl.ds(row0, ROWS_PER)],
                  acc_sh.at[pl.ds(row0, ROWS_PER)])
  pltpu.sync_copy(src_hbm.at[s], src_v)
  pltpu.sync_copy(dst_hbm.at[s], dst_v)
  plsc.subcore_barrier()

  pltpu.async_copy(tbl_sh.at[src_v.at[0]], buf0, sem0)

  def body(t, carry):
    g0 = 2 * t
    pltpu.async_copy(tbl_sh.at[src_v.at[g0 + 1]], buf1, sem1)
    pltpu.make_async_copy(tbl_sh.at[src_v.at[g0]], buf0, sem0).wait()
    pltpu.sync_copy(buf0, acc_sh.at[dst_v.at[g0]], add=True)
    pltpu.async_copy(tbl_sh.at[src_v.at[g0 + 2]], buf0, sem0)
    pltpu.make_async_copy(tbl_sh.at[src_v.at[g0 + 1]], buf1, sem1).wait()
    pltpu.sync_copy(buf1, acc_sh.at[dst_v.at[g0 + 1]], add=True)
    return carry

  lax.fori_loop(0, G2 // 2, body, 0)
  pltpu.make_async_copy(tbl_sh.at[src_v.at[G2]], buf0, sem0).wait()
  plsc.subcore_barrier()

  pltpu.sync_copy(rc_hbm.at[pl.ds(row0, ROWS_PER)], rc_v)
  pltpu.sync_copy(hr_hbm.at[pl.ds(row0, ROWS_PER), pl.ds(col0, DH2)], hr_v)
  pltpu.sync_copy(b2_hbm.at[pl.ds(c * DH2, DH2)], b2_v)
  pltpu.sync_copy(acc_sh.at[pl.ds(row0, ROWS_PER)], out_v)
  b2vec = b2_v[...]

  def eblk(q, carry):
    rc16 = rc_v[pl.ds(q * 16, 16)]
    for i in range(16):
      r = q * 16 + i
      out_v[r, :] = out_v[r, :] * rc16[i] + b2vec + hr_v[r, :]
    return carry

  lax.fori_loop(0, ROWS_PER // 16, eblk, 0)
  pltpu.sync_copy(out_v, out_hbm.at[pl.ds(row0, ROWS_PER), pl.ds(col0, DH2)])


def _tc1_body(x_ref, wl_ref, wr_ref, y1_ref, xr_ref):
  xb = x_ref[...]
  yl = jnp.dot(xb, wl_ref[...], preferred_element_type=jnp.float32)
  col = lax.broadcasted_iota(jnp.int32, (BLK, D1 - HIDDEN), 1)
  cnt_col = jnp.where(col == 0, 1.0, 0.0).astype(jnp.float32)
  y1_ref[...] = jnp.concatenate([yl, cnt_col], axis=1)
  xr_ref[...] = jnp.dot(xb, wr_ref[...], preferred_element_type=jnp.float32)


def _tc2_body(p_ref, xr_ref, b_ref, wl2_ref, wr2_ref, y2_ref, hr_ref, rc_ref):
  agg = p_ref[...]
  col = lax.broadcasted_iota(jnp.int32, (BLK, D1), 1)
  cnt = jnp.sum(jnp.where(col == HIDDEN, agg, 0.0), axis=1, keepdims=True)
  rc = 1.0 / jnp.maximum(cnt, 1.0)
  h = jnp.maximum(agg * rc + b_ref[...] + xr_ref[...], 0.0)
  y2_ref[...] = jnp.dot(h, wl2_ref[...], preferred_element_type=jnp.float32)
  hr_ref[...] = jnp.dot(h, wr2_ref[...], preferred_element_type=jnp.float32)
  rc_ref[...] = rc


def kernel(x, edge_index, W1l, b1, W1r, W2l, b2, W2r):
  f32 = jnp.float32
  grid = NPAD // BLK

  src = edge_index[0].astype(jnp.int32)
  dst = edge_index[1].astype(jnp.int32)
  pad = E_PAD - N_EDGES
  src_p = jnp.concatenate(
      [src, jnp.zeros((pad,), jnp.int32)]).reshape(NSUB, EPT)
  src_p = jnp.concatenate([src_p, jnp.zeros((NSUB, SLAB2), jnp.int32)],
                          axis=1)
  sink = SINK + jnp.arange(pad, dtype=jnp.int32) % (NPAD - SINK)
  dst_p = jnp.concatenate([dst, sink]).reshape(NSUB, EPT)
  src3a = src_p.reshape(NSUB, G1 + KK2, CHUNK)
  src3b = src_p.reshape(NSUB, G2 + 1, SLAB2)
  dst3a = dst_p.reshape(NSUB, G1, CHUNK)
  dst3b = dst_p.reshape(NSUB, G2, SLAB2)
  w1l_t = W1l.T
  w1r_t = jnp.zeros((IN_FEATS, D1), f32).at[:, :HIDDEN].set(W1r.T)
  b1p = jnp.zeros((1, D1), f32).at[0, :HIDDEN].set(b1)
  w2l_t = jnp.zeros((D1, OUT_FEATS), f32).at[:HIDDEN].set(W2l.T)
  w2r_t = jnp.zeros((D1, OUT_FEATS), f32).at[:HIDDEN].set(W2r.T)
  zeros1 = jnp.zeros((NPAD, DH1), f32)
  zeros2 = jnp.zeros((NPAD, DH2), f32)

  y1, xr1 = pl.pallas_call(
      _tc1_body,
      grid=(grid,),
      in_specs=[
          pl.BlockSpec((BLK, IN_FEATS), lambda i: (i, 0)),
          pl.BlockSpec((IN_FEATS, HIDDEN), lambda i: (0, 0)),
          pl.BlockSpec((IN_FEATS, D1), lambda i: (0, 0)),
      ],
      out_specs=[
          pl.BlockSpec((BLK, D1), lambda i: (i, 0)),
          pl.BlockSpec((BLK, D1), lambda i: (i, 0)),
      ],
      out_shape=[
          jax.ShapeDtypeStruct((NPAD, D1), f32),
          jax.ShapeDtypeStruct((NPAD, D1), f32),
      ],
  )(x, w1l_t, w1r_t)

  p1 = _sc_segsum_l1(y1, src3a, dst3a, zeros1)

  y2, hr2, rc = pl.pallas_call(
      _tc2_body,
      grid=(grid,),
      in_specs=[
          pl.BlockSpec((BLK, D1), lambda i: (i, 0)),
          pl.BlockSpec((BLK, D1), lambda i: (i, 0)),
          pl.BlockSpec((1, D1), lambda i: (0, 0)),
          pl.BlockSpec((D1, OUT_FEATS), lambda i: (0, 0)),
          pl.BlockSpec((D1, OUT_FEATS), lambda i: (0, 0)),
      ],
      out_specs=[
          pl.BlockSpec((BLK, D2), lambda i: (i, 0)),
          pl.BlockSpec((BLK, D2), lambda i: (i, 0)),
          pl.BlockSpec((BLK, 1), lambda i: (i, 0)),
      ],
      out_shape=[
          jax.ShapeDtypeStruct((NPAD, D2), f32),
          jax.ShapeDtypeStruct((NPAD, D2), f32),
          jax.ShapeDtypeStruct((NPAD, 1), f32),
      ],
  )(p1, xr1, b1p, w2l_t, w2r_t)

  out = _sc_segsum_l2(y2, src3b, dst3b, zeros2, rc.reshape(NPAD), hr2, b2)

  return out[:N_NODES]
